# baseline (speedup 1.0000x reference)
import functools

import jax
import jax.numpy as jnp
from jax import lax
from jax.experimental import pallas as pl
from jax.experimental.pallas import tpu as pltpu
from jax.experimental.pallas import tpu_sc as plsc

_E = 8
_H = 768
_I = 1536
_T = 2048
_BM = 256
_APAD = _T * 2 + _E * _BM
_NBLK = _APAD // _BM
_NW = 16
_TPW = _T // _NW
_LANES = 16


def _router_body(x_ref, rw_ref, logits_ref):
    logits_ref[...] = jnp.dot(x_ref[...], rw_ref[...],
                              preferred_element_type=jnp.float32)


def _splat(x):
    return jnp.full((_LANES,), x, jnp.int32)


def _dyn_gather(vec, idx):
    dnums = lax.GatherDimensionNumbers(
        offset_dims=(), collapsed_slice_dims=(0,), start_index_map=(0,))
    return lax.gather(vec, idx[:, None], dnums, slice_sizes=(1,),
                      mode=lax.GatherScatterMode.PROMISE_IN_BOUNDS)


def _route_body(logits_hbm, e1_hbm, e2_hbm, w0_hbm, w1_hbm, cnts_hbm, slab,
                e1b, e2b, w0b, w1b, cntv):
    nw = 32
    tpw = _T // nw
    wid = lax.axis_index("s") * 2 + lax.axis_index("c")
    lanes = lax.iota(jnp.int32, _LANES)
    zero_i = jnp.zeros((_LANES,), jnp.int32)

    pltpu.sync_copy(logits_hbm.at[pl.ds(wid * (tpw * _E), tpw * _E)], slab)
    cnt = zero_i
    for c in range(tpw // _LANES):
        tloc = lanes + c * _LANES
        m1 = jnp.full((_LANES,), -jnp.inf, jnp.float32)
        m2 = m1
        i1 = zero_i
        i2 = zero_i
        for e in range(_E):
            v = plsc.load_gather(slab, [tloc * _E + e])
            gt1 = v > m1
            gt2 = v > m2
            i2 = jnp.where(gt1, i1, jnp.where(gt2, e, i2))
            m2 = jnp.where(gt1, m1, jnp.where(gt2, v, m2))
            i1 = jnp.where(gt1, e, i1)
            m1 = jnp.where(gt1, v, m1)
        w0 = 1.0 / (1.0 + jnp.exp(m2 - m1))
        sl = pl.ds(c * _LANES, _LANES)
        e1b[sl] = i1
        e2b[sl] = i2
        w0b[sl] = w0
        w1b[sl] = 1.0 - w0
        for e in range(_E):
            pc = plsc.all_reduce_population_count(i1 == e)
            pc = pc + plsc.all_reduce_population_count(i2 == e)
            cnt = cnt + jnp.where(lanes == e, pc, 0)
    cntv[...] = cnt
    pltpu.sync_copy(e1b, e1_hbm.at[pl.ds(wid * tpw, tpw)])
    pltpu.sync_copy(e2b, e2_hbm.at[pl.ds(wid * tpw, tpw)])
    pltpu.sync_copy(w0b, w0_hbm.at[pl.ds(wid * tpw, tpw)])
    pltpu.sync_copy(w1b, w1_hbm.at[pl.ds(wid * tpw, tpw)])
    pltpu.sync_copy(cntv, cnts_hbm.at[pl.ds(wid * _LANES, _LANES)])


def _dispatch_body(cnts_hbm, e1_hbm, e2_hbm, w0_hbm, w1_hbm, x_hbm, xs_hbm,
                   w2d_hbm, pos0_hbm, pos1_hbm, be_hbm, ba_hbm, callv, e1b,
                   e2b, w0b, w1b, idxb, p0b, p1b, zidxb, xbuf, wbuf, zbuf,
                   bev, bav, sem):
    nw = 32
    tpw = _T // nw
    nck = tpw // _LANES
    wid = lax.axis_index("s") * 2 + lax.axis_index("c")
    lanes = lax.iota(jnp.int32, _LANES)
    zero_i = jnp.zeros((_LANES,), jnp.int32)

    pltpu.sync_copy(cnts_hbm, callv)
    pltpu.sync_copy(e1_hbm.at[pl.ds(wid * tpw, tpw)], e1b)
    pltpu.sync_copy(e2_hbm.at[pl.ds(wid * tpw, tpw)], e2b)
    pltpu.sync_copy(w0_hbm.at[pl.ds(wid * tpw, tpw)], w0b)
    pltpu.sync_copy(w1_hbm.at[pl.ds(wid * tpw, tpw)], w1b)

    acc = zero_i
    before = zero_i
    for w2 in range(nw):
        row = callv[pl.ds(w2 * _LANES, _LANES)]
        before = jnp.where(_splat(w2) == _splat(wid), acc, before)
        acc = acc + row
    tot = acc
    pad_tot = ((tot + (_BM - 1)) // _BM) * _BM
    off_pad = plsc.cumsum(pad_tot) - pad_tot
    base_vec = off_pad + before
    nb = pad_tot // _BM
    ends = plsc.cumsum(nb)
    used = jnp.sum(jnp.where(lanes < _E, nb, 0))

    @pl.when(wid == 0)
    def _blocks():
        last_e = jnp.sum(jnp.where((ends <= used - 1) & (lanes < _E), 1, 0))
        for c in range(48 // _LANES):
            bv = lanes + c * _LANES
            raw = zero_i
            for e in range(_E):
                ends_e = _dyn_gather(ends, _splat(e))
                raw = raw + jnp.where(ends_e <= bv, 1, 0)
            active = bv < used
            bev[pl.ds(c * _LANES, _LANES)] = jnp.where(active, raw, last_e)
            bav[pl.ds(c * _LANES, _LANES)] = active.astype(jnp.int32)
        pltpu.sync_copy(bev, be_hbm)
        pltpu.sync_copy(bav, ba_hbm)

    cnt_run = zero_i
    for cl in range(nck):
        lsl = pl.ds(cl * _LANES, _LANES)
        for s in range(2):
            ev = (e1b if s == 0 else e2b)[lsl]
            g = _dyn_gather(cnt_run, ev)
            prefix = zero_i
            ccnt = zero_i
            for e in range(_E):
                selm = ev == e
                pcs = plsc.cumsum(selm.astype(jnp.int32))
                prefix = jnp.where(selm, pcs - 1, prefix)
                ccnt = ccnt + jnp.where(
                    lanes == e, plsc.all_reduce_population_count(selm), 0)
            pos = _dyn_gather(base_vec, ev) + g + prefix
            cnt_run = cnt_run + ccnt
            idxb[cl * 2 + s, :] = pos
            if s == 0:
                p0b[lsl] = pos
            else:
                p1b[lsl] = pos
    pltpu.sync_copy(p0b, pos0_hbm.at[pl.ds(wid * tpw, tpw)])
    pltpu.sync_copy(p1b, pos1_hbm.at[pl.ds(wid * tpw, tpw)])

    for k in range(_LANES):
        for kk in range(128 // _LANES):
            zbuf[k, pl.ds(kk * _LANES, _LANES)] = jnp.zeros((_LANES,),
                                                            jnp.float32)
    for k in range(_BM // _LANES):
        j = lanes + k * _LANES
        dump = _APAD + lanes
        myexp = _splat(jnp.minimum(wid, _E - 1))
        cnt_e = _dyn_gather(tot, myexp)
        pad_e = _dyn_gather(pad_tot, myexp)
        off_e = _dyn_gather(off_pad, myexp)
        padidx = jnp.where(cnt_e == pad_e, dump,
                           off_e + jnp.minimum(cnt_e + j, pad_e - 1))
        zidxb[k, :] = jnp.where(_splat(wid) < _E, padidx, dump)
    for k in range(_BM // _LANES):
        pltpu.async_copy(zbuf, w2d_hbm.at[zidxb.at[k]], sem).wait()

    for cl in range(nck):
        pltpu.sync_copy(x_hbm.at[pl.ds(wid * tpw + cl * _LANES, _LANES)],
                        xbuf)
        for s in range(2):
            wv = (w0b if s == 0 else w1b)[pl.ds(cl * _LANES, _LANES)]
            for r in range(_LANES):
                wr = _dyn_gather(wv, _splat(r))
                wbuf[r, pl.ds(0, _LANES)] = wr
            da = pltpu.async_copy(xbuf, xs_hbm.at[idxb.at[cl * 2 + s]], sem)
            db = pltpu.async_copy(wbuf, w2d_hbm.at[idxb.at[cl * 2 + s]], sem)
            da.wait()
            db.wait()


def _gmm_body(be_ref, ba_ref, xs_ref, wg_ref, wu_ref, wd_ref, wcol_ref,
              out_ref):
    i = pl.program_id(0)

    @pl.when(ba_ref[i] == 1)
    def _active():
        xb = xs_ref[...].astype(jnp.bfloat16)
        g = jnp.dot(xb, wg_ref[0], preferred_element_type=jnp.float32)
        u = jnp.dot(xb, wu_ref[0], preferred_element_type=jnp.float32)
        act = (g * jax.nn.sigmoid(g) * u).astype(jnp.bfloat16)
        ob = jnp.dot(act, wd_ref[0], preferred_element_type=jnp.float32)
        out_ref[...] = ob * wcol_ref[:, 0:1]

    @pl.when(ba_ref[i] == 0)
    def _inactive():
        out_ref[...] = jnp.zeros_like(out_ref)


def _combine_body(os_hbm, p03_hbm, p13_hbm, out_hbm, idxA, idxB, bufA, bufB,
                  sem):
    nw2 = 32
    tpw = _T // nw2
    ck = _LANES
    wid = lax.axis_index("s") * 2 + lax.axis_index("c")
    pltpu.sync_copy(p03_hbm.at[wid], idxA)
    pltpu.sync_copy(p13_hbm.at[wid], idxB)
    for c in range(tpw // ck):
        da = pltpu.async_copy(os_hbm.at[idxA.at[c]], bufA, sem)
        db = pltpu.async_copy(os_hbm.at[idxB.at[c]], bufB, sem)
        da.wait()
        db.wait()

        def _add(r, carry):
            for k in range(_H // _LANES):
                sl = pl.ds(k * _LANES, _LANES)
                bufA[r, sl] = bufA[r, sl] + bufB[r, sl]
            return carry
        lax.fori_loop(0, ck, _add, 0)
        pltpu.sync_copy(bufA, out_hbm.at[pl.ds(wid * tpw + c * ck, ck)])


_SC_PARAMS = pltpu.CompilerParams(needs_layout_passes=False)
_MESH1 = plsc.VectorSubcoreMesh(core_axis_name="c", subcore_axis_name="s",
                                num_cores=1)
_MESH2 = plsc.VectorSubcoreMesh(core_axis_name="c", subcore_axis_name="s",
                                num_cores=2)


def _make_router():
    return pl.pallas_call(
        _router_body,
        in_specs=[pl.BlockSpec((_T, _H), lambda: (0, 0)),
                  pl.BlockSpec((_H, _E), lambda: (0, 0))],
        out_specs=pl.BlockSpec((_T, _E), lambda: (0, 0)),
        out_shape=jax.ShapeDtypeStruct((_T, _E), jnp.float32),
    )


def _make_route():
    return pl.kernel(
        _route_body,
        compiler_params=_SC_PARAMS,
        out_type=[
            jax.ShapeDtypeStruct((_T,), jnp.int32),
            jax.ShapeDtypeStruct((_T,), jnp.int32),
            jax.ShapeDtypeStruct((_T,), jnp.float32),
            jax.ShapeDtypeStruct((_T,), jnp.float32),
            jax.ShapeDtypeStruct((32 * _LANES,), jnp.int32),
        ],
        mesh=_MESH2,
        scratch_types=[
            pltpu.VMEM((_T // 32 * _E,), jnp.float32),
            pltpu.VMEM((64,), jnp.int32),
            pltpu.VMEM((64,), jnp.int32),
            pltpu.VMEM((64,), jnp.float32),
            pltpu.VMEM((64,), jnp.float32),
            pltpu.VMEM((_LANES,), jnp.int32),
        ],
    )


def _make_dispatch():
    return pl.kernel(
        _dispatch_body,
        compiler_params=_SC_PARAMS,
        out_type=[
            jax.ShapeDtypeStruct((_APAD, _H), jnp.float32),
            jax.ShapeDtypeStruct((_APAD + 128, 128), jnp.float32),
            jax.ShapeDtypeStruct((_T,), jnp.int32),
            jax.ShapeDtypeStruct((_T,), jnp.int32),
            jax.ShapeDtypeStruct((48,), jnp.int32),
            jax.ShapeDtypeStruct((48,), jnp.int32),
        ],
        mesh=_MESH2,
        scratch_types=[
            pltpu.VMEM((32 * _LANES,), jnp.int32),
            pltpu.VMEM((64,), jnp.int32),
            pltpu.VMEM((64,), jnp.int32),
            pltpu.VMEM((64,), jnp.float32),
            pltpu.VMEM((64,), jnp.float32),
            pltpu.VMEM((8, _LANES), jnp.int32),
            pltpu.VMEM((64,), jnp.int32),
            pltpu.VMEM((64,), jnp.int32),
            pltpu.VMEM((_BM // _LANES, _LANES), jnp.int32),
            pltpu.VMEM((_LANES, _H), jnp.float32),
            pltpu.VMEM((_LANES, 128), jnp.float32),
            pltpu.VMEM((_LANES, 128), jnp.float32),
            pltpu.VMEM((48,), jnp.int32),
            pltpu.VMEM((48,), jnp.int32),
            pltpu.SemaphoreType.DMA,
        ],
    )


def _make_gmm():
    return pl.pallas_call(
        _gmm_body,
        grid_spec=pltpu.PrefetchScalarGridSpec(
            num_scalar_prefetch=2,
            grid=(_NBLK,),
            in_specs=[
                pl.BlockSpec((_BM, _H), lambda i, be, ba: (i, 0)),
                pl.BlockSpec((1, _H, _I), lambda i, be, ba: (be[i], 0, 0)),
                pl.BlockSpec((1, _H, _I), lambda i, be, ba: (be[i], 0, 0)),
                pl.BlockSpec((1, _I, _H), lambda i, be, ba: (be[i], 0, 0)),
                pl.BlockSpec((_BM, 128), lambda i, be, ba: (i, 0)),
            ],
            out_specs=pl.BlockSpec((_BM, _H), lambda i, be, ba: (i, 0)),
        ),
        out_shape=jax.ShapeDtypeStruct((_APAD, _H), jnp.float32),
    )


def _make_combine():
    return pl.kernel(
        _combine_body,
        compiler_params=_SC_PARAMS,
        out_type=jax.ShapeDtypeStruct((_T, _H), jnp.float32),
        mesh=_MESH2,
        scratch_types=[
            pltpu.VMEM((4, _LANES), jnp.int32),
            pltpu.VMEM((4, _LANES), jnp.int32),
            pltpu.VMEM((_LANES, _H), jnp.float32),
            pltpu.VMEM((_LANES, _H), jnp.float32),
            pltpu.SemaphoreType.DMA,
        ],
    )


def kernel(hidden_states, router_w, w_gate, w_up, w_down):
    shape = hidden_states.shape
    x = hidden_states.reshape(-1, _H)
    wg = w_gate.astype(jnp.bfloat16)
    wu = w_up.astype(jnp.bfloat16)
    wd = w_down.astype(jnp.bfloat16)

    logits = _make_router()(x, router_w)
    e1, e2, w0, w1, cnts = _make_route()(logits.reshape(-1))
    x_sorted, w2d, pos0, pos1, blk_e, blk_a = _make_dispatch()(
        cnts, e1, e2, w0, w1, x)
    out_sorted = _make_gmm()(blk_e, blk_a, x_sorted, wg, wu, wd,
                             w2d[:_APAD])
    p03 = pos0.reshape(32, 4, _LANES)
    p13 = pos1.reshape(32, 4, _LANES)
    out = _make_combine()(out_sorted, p03, p13)
    return out.reshape(shape), logits


# speedup vs baseline: 1.1960x; 1.1960x over previous
import functools

import jax
import jax.numpy as jnp
from jax import lax
from jax.experimental import pallas as pl
from jax.experimental.pallas import tpu as pltpu
from jax.experimental.pallas import tpu_sc as plsc

_E = 8
_H = 768
_I = 1536
_T = 2048
_BM = 128
_APAD = _T * 2 + _E * _BM
_NBLK = _APAD // _BM
_NW = 16
_TPW = _T // _NW
_LANES = 16


def _router_body(x_ref, rw_ref, logits_ref):
    logits_ref[...] = jnp.dot(x_ref[...], rw_ref[...],
                              preferred_element_type=jnp.float32)


def _splat(x):
    return jnp.full((_LANES,), x, jnp.int32)


def _dyn_gather(vec, idx):
    dnums = lax.GatherDimensionNumbers(
        offset_dims=(), collapsed_slice_dims=(0,), start_index_map=(0,))
    return lax.gather(vec, idx[:, None], dnums, slice_sizes=(1,),
                      mode=lax.GatherScatterMode.PROMISE_IN_BOUNDS)


def _route_body(logits_hbm, e1_hbm, e2_hbm, w0_hbm, w1_hbm, cnts_hbm, slab,
                e1b, e2b, w0b, w1b, cntv):
    nw = 32
    tpw = _T // nw
    wid = lax.axis_index("s") * 2 + lax.axis_index("c")
    lanes = lax.iota(jnp.int32, _LANES)
    zero_i = jnp.zeros((_LANES,), jnp.int32)

    pltpu.sync_copy(logits_hbm.at[pl.ds(wid * (tpw * _E), tpw * _E)], slab)
    cnt = zero_i
    for c in range(tpw // _LANES):
        tloc = lanes + c * _LANES
        m1 = jnp.full((_LANES,), -jnp.inf, jnp.float32)
        m2 = m1
        i1 = zero_i
        i2 = zero_i
        for e in range(_E):
            v = plsc.load_gather(slab, [tloc * _E + e])
            gt1 = v > m1
            gt2 = v > m2
            i2 = jnp.where(gt1, i1, jnp.where(gt2, e, i2))
            m2 = jnp.where(gt1, m1, jnp.where(gt2, v, m2))
            i1 = jnp.where(gt1, e, i1)
            m1 = jnp.where(gt1, v, m1)
        w0 = 1.0 / (1.0 + jnp.exp(m2 - m1))
        sl = pl.ds(c * _LANES, _LANES)
        e1b[sl] = i1
        e2b[sl] = i2
        w0b[sl] = w0
        w1b[sl] = 1.0 - w0
        for e in range(_E):
            pc = plsc.all_reduce_population_count(i1 == e)
            pc = pc + plsc.all_reduce_population_count(i2 == e)
            cnt = cnt + jnp.where(lanes == e, pc, 0)
    cntv[...] = cnt
    pltpu.sync_copy(e1b, e1_hbm.at[pl.ds(wid * tpw, tpw)])
    pltpu.sync_copy(e2b, e2_hbm.at[pl.ds(wid * tpw, tpw)])
    pltpu.sync_copy(w0b, w0_hbm.at[pl.ds(wid * tpw, tpw)])
    pltpu.sync_copy(w1b, w1_hbm.at[pl.ds(wid * tpw, tpw)])
    pltpu.sync_copy(cntv, cnts_hbm.at[pl.ds(wid * _LANES, _LANES)])


def _dispatch_body(cnts_hbm, e1_hbm, e2_hbm, w0_hbm, w1_hbm, x_hbm, xs_hbm,
                   w2d_hbm, pos0_hbm, pos1_hbm, be_hbm, ba_hbm, callv, e1b,
                   e2b, w0b, w1b, idxb, p0b, p1b, zidxb, xbuf, wbuf, zbuf,
                   bev, bav, sem, rsem):
    nw = 32
    tpw = _T // nw
    nck = tpw // _LANES
    wid = lax.axis_index("s") * 2 + lax.axis_index("c")
    lanes = lax.iota(jnp.int32, _LANES)
    zero_i = jnp.zeros((_LANES,), jnp.int32)

    pltpu.sync_copy(cnts_hbm, callv)
    pltpu.sync_copy(e1_hbm.at[pl.ds(wid * tpw, tpw)], e1b)
    pltpu.sync_copy(e2_hbm.at[pl.ds(wid * tpw, tpw)], e2b)
    pltpu.sync_copy(w0_hbm.at[pl.ds(wid * tpw, tpw)], w0b)
    pltpu.sync_copy(w1_hbm.at[pl.ds(wid * tpw, tpw)], w1b)

    acc = zero_i
    before = zero_i
    for w2 in range(nw):
        row = callv[pl.ds(w2 * _LANES, _LANES)]
        before = jnp.where(_splat(w2) == _splat(wid), acc, before)
        acc = acc + row
    tot = acc
    pad_tot = ((tot + (_BM - 1)) // _BM) * _BM
    off_pad = plsc.cumsum(pad_tot) - pad_tot
    base_vec = off_pad + before
    nb = pad_tot // _BM
    ends = plsc.cumsum(nb)
    used = jnp.sum(jnp.where(lanes < _E, nb, 0))

    @pl.when(wid == 0)
    def _blocks():
        last_e = jnp.sum(jnp.where((ends <= used - 1) & (lanes < _E), 1, 0))
        for c in range(48 // _LANES):
            bv = lanes + c * _LANES
            raw = zero_i
            for e in range(_E):
                ends_e = _dyn_gather(ends, _splat(e))
                raw = raw + jnp.where(ends_e <= bv, 1, 0)
            active = bv < used
            bev[pl.ds(c * _LANES, _LANES)] = jnp.where(active, raw, last_e)
            bav[pl.ds(c * _LANES, _LANES)] = active.astype(jnp.int32)
        pltpu.sync_copy(bev, be_hbm)
        pltpu.sync_copy(bav, ba_hbm)

    cnt_run = zero_i
    for cl in range(nck):
        lsl = pl.ds(cl * _LANES, _LANES)
        for s in range(2):
            ev = (e1b if s == 0 else e2b)[lsl]
            g = _dyn_gather(cnt_run, ev)
            prefix = zero_i
            ccnt = zero_i
            for e in range(_E):
                selm = ev == e
                pcs = plsc.cumsum(selm.astype(jnp.int32))
                prefix = jnp.where(selm, pcs - 1, prefix)
                ccnt = ccnt + jnp.where(
                    lanes == e, plsc.all_reduce_population_count(selm), 0)
            pos = _dyn_gather(base_vec, ev) + g + prefix
            cnt_run = cnt_run + ccnt
            idxb[cl * 2 + s, :] = pos
            if s == 0:
                p0b[lsl] = pos
            else:
                p1b[lsl] = pos
    pltpu.sync_copy(p0b, pos0_hbm.at[pl.ds(wid * tpw, tpw)])
    pltpu.sync_copy(p1b, pos1_hbm.at[pl.ds(wid * tpw, tpw)])

    for k in range(_LANES):
        for kk in range(128 // _LANES):
            zbuf[k, pl.ds(kk * _LANES, _LANES)] = jnp.zeros((_LANES,),
                                                            jnp.float32)
    for k in range(_BM // _LANES):
        j = lanes + k * _LANES
        dump = _APAD + lanes
        myexp = _splat(jnp.minimum(wid, _E - 1))
        cnt_e = _dyn_gather(tot, myexp)
        pad_e = _dyn_gather(pad_tot, myexp)
        off_e = _dyn_gather(off_pad, myexp)
        padidx = jnp.where(cnt_e == pad_e, dump,
                           off_e + jnp.minimum(cnt_e + j, pad_e - 1))
        zidxb[k, :] = jnp.where(_splat(wid) < _E, padidx, dump)
    wdescs = []
    for k in range(_BM // _LANES):
        wdescs.append(pltpu.async_copy(zbuf, w2d_hbm.at[zidxb.at[k]], sem))

    rdescs = []
    for cl in range(nck):
        rdescs.append(pltpu.async_copy(
            x_hbm.at[pl.ds(wid * tpw + cl * _LANES, _LANES)], xbuf.at[cl],
            rsem))
    for cl in range(nck):
        rdescs[cl].wait()
        for s in range(2):
            wv = (w0b if s == 0 else w1b)[pl.ds(cl * _LANES, _LANES)]
            for r in range(_LANES):
                wr = _dyn_gather(wv, _splat(r))
                wbuf[cl * 2 + s, r, pl.ds(0, _LANES)] = wr
            wdescs.append(pltpu.async_copy(
                xbuf.at[cl], xs_hbm.at[idxb.at[cl * 2 + s]], sem))
            wdescs.append(pltpu.async_copy(
                wbuf.at[cl * 2 + s], w2d_hbm.at[idxb.at[cl * 2 + s]], sem))
    for d in wdescs:
        d.wait()


def _gmm_body(be_ref, ba_ref, xs_ref, wg_ref, wu_ref, wd_ref, wcol_ref,
              out_ref):
    i = pl.program_id(0)

    @pl.when(ba_ref[i] == 1)
    def _active():
        xb = xs_ref[...].astype(jnp.bfloat16)
        wg16 = wg_ref[0].astype(jnp.bfloat16)
        wu16 = wu_ref[0].astype(jnp.bfloat16)
        wd16 = wd_ref[0].astype(jnp.bfloat16)
        g = jnp.dot(xb, wg16, preferred_element_type=jnp.float32)
        u = jnp.dot(xb, wu16, preferred_element_type=jnp.float32)
        act = (g * jax.nn.sigmoid(g) * u).astype(jnp.bfloat16)
        ob = jnp.dot(act, wd16, preferred_element_type=jnp.float32)
        out_ref[...] = ob * wcol_ref[:, 0:1]

    @pl.when(ba_ref[i] == 0)
    def _inactive():
        out_ref[...] = jnp.zeros_like(out_ref)


def _combine_body(os_hbm, p03_hbm, p13_hbm, out_hbm, idxA, idxB, bufA, bufB,
                  sem):
    nw2 = 32
    tpw = _T // nw2
    ck = _LANES
    wid = lax.axis_index("s") * 2 + lax.axis_index("c")
    pltpu.sync_copy(p03_hbm.at[wid], idxA)
    pltpu.sync_copy(p13_hbm.at[wid], idxB)
    for c in range(tpw // ck):
        da = pltpu.async_copy(os_hbm.at[idxA.at[c]], bufA, sem)
        db = pltpu.async_copy(os_hbm.at[idxB.at[c]], bufB, sem)
        da.wait()
        db.wait()

        def _add(r, carry):
            for k in range(_H // _LANES):
                sl = pl.ds(k * _LANES, _LANES)
                bufA[r, sl] = bufA[r, sl] + bufB[r, sl]
            return carry
        lax.fori_loop(0, ck, _add, 0)
        pltpu.sync_copy(bufA, out_hbm.at[pl.ds(wid * tpw + c * ck, ck)])


_SC_PARAMS = pltpu.CompilerParams(needs_layout_passes=False)
_MESH1 = plsc.VectorSubcoreMesh(core_axis_name="c", subcore_axis_name="s",
                                num_cores=1)
_MESH2 = plsc.VectorSubcoreMesh(core_axis_name="c", subcore_axis_name="s",
                                num_cores=2)


def _make_router():
    return pl.pallas_call(
        _router_body,
        in_specs=[pl.BlockSpec((_T, _H), lambda: (0, 0)),
                  pl.BlockSpec((_H, _E), lambda: (0, 0))],
        out_specs=pl.BlockSpec((_T, _E), lambda: (0, 0)),
        out_shape=jax.ShapeDtypeStruct((_T, _E), jnp.float32),
    )


def _make_route():
    return pl.kernel(
        _route_body,
        compiler_params=_SC_PARAMS,
        out_type=[
            jax.ShapeDtypeStruct((_T,), jnp.int32),
            jax.ShapeDtypeStruct((_T,), jnp.int32),
            jax.ShapeDtypeStruct((_T,), jnp.float32),
            jax.ShapeDtypeStruct((_T,), jnp.float32),
            jax.ShapeDtypeStruct((32 * _LANES,), jnp.int32),
        ],
        mesh=_MESH2,
        scratch_types=[
            pltpu.VMEM((_T // 32 * _E,), jnp.float32),
            pltpu.VMEM((64,), jnp.int32),
            pltpu.VMEM((64,), jnp.int32),
            pltpu.VMEM((64,), jnp.float32),
            pltpu.VMEM((64,), jnp.float32),
            pltpu.VMEM((_LANES,), jnp.int32),
        ],
    )


def _make_dispatch():
    return pl.kernel(
        _dispatch_body,
        compiler_params=_SC_PARAMS,
        out_type=[
            jax.ShapeDtypeStruct((_APAD, _H), jnp.float32),
            jax.ShapeDtypeStruct((_APAD + 128, 128), jnp.float32),
            jax.ShapeDtypeStruct((_T,), jnp.int32),
            jax.ShapeDtypeStruct((_T,), jnp.int32),
            jax.ShapeDtypeStruct((48,), jnp.int32),
            jax.ShapeDtypeStruct((48,), jnp.int32),
        ],
        mesh=_MESH2,
        scratch_types=[
            pltpu.VMEM((32 * _LANES,), jnp.int32),
            pltpu.VMEM((64,), jnp.int32),
            pltpu.VMEM((64,), jnp.int32),
            pltpu.VMEM((64,), jnp.float32),
            pltpu.VMEM((64,), jnp.float32),
            pltpu.VMEM((8, _LANES), jnp.int32),
            pltpu.VMEM((64,), jnp.int32),
            pltpu.VMEM((64,), jnp.int32),
            pltpu.VMEM((_BM // _LANES, _LANES), jnp.int32),
            pltpu.VMEM((4, _LANES, _H), jnp.float32),
            pltpu.VMEM((8, _LANES, 128), jnp.float32),
            pltpu.VMEM((_LANES, 128), jnp.float32),
            pltpu.VMEM((48,), jnp.int32),
            pltpu.VMEM((48,), jnp.int32),
            pltpu.SemaphoreType.DMA,
            pltpu.SemaphoreType.DMA,
        ],
    )


def _make_gmm():
    return pl.pallas_call(
        _gmm_body,
        grid_spec=pltpu.PrefetchScalarGridSpec(
            num_scalar_prefetch=2,
            grid=(_NBLK,),
            in_specs=[
                pl.BlockSpec((_BM, _H), lambda i, be, ba: (i, 0)),
                pl.BlockSpec((1, _H, _I), lambda i, be, ba: (be[i], 0, 0)),
                pl.BlockSpec((1, _H, _I), lambda i, be, ba: (be[i], 0, 0)),
                pl.BlockSpec((1, _I, _H), lambda i, be, ba: (be[i], 0, 0)),
                pl.BlockSpec((_BM, 128), lambda i, be, ba: (i, 0)),
            ],
            out_specs=pl.BlockSpec((_BM, _H), lambda i, be, ba: (i, 0)),
        ),
        out_shape=jax.ShapeDtypeStruct((_APAD, _H), jnp.float32),
    )


def _make_combine():
    return pl.kernel(
        _combine_body,
        compiler_params=_SC_PARAMS,
        out_type=jax.ShapeDtypeStruct((_T, _H), jnp.float32),
        mesh=_MESH2,
        scratch_types=[
            pltpu.VMEM((4, _LANES), jnp.int32),
            pltpu.VMEM((4, _LANES), jnp.int32),
            pltpu.VMEM((_LANES, _H), jnp.float32),
            pltpu.VMEM((_LANES, _H), jnp.float32),
            pltpu.SemaphoreType.DMA,
        ],
    )


def kernel(hidden_states, router_w, w_gate, w_up, w_down):
    shape = hidden_states.shape
    x = hidden_states.reshape(-1, _H)
    logits = _make_router()(x, router_w)
    e1, e2, w0, w1, cnts = _make_route()(logits.reshape(-1))
    x_sorted, w2d, pos0, pos1, blk_e, blk_a = _make_dispatch()(
        cnts, e1, e2, w0, w1, x)
    out_sorted = _make_gmm()(blk_e, blk_a, x_sorted, w_gate, w_up, w_down,
                             w2d[:_APAD])
    p03 = pos0.reshape(32, 4, _LANES)
    p13 = pos1.reshape(32, 4, _LANES)
    out = _make_combine()(out_sorted, p03, p13)
    return out.reshape(shape), logits


# speedup vs baseline: 1.2244x; 1.0237x over previous
import functools

import jax
import jax.numpy as jnp
from jax import lax
from jax.experimental import pallas as pl
from jax.experimental.pallas import tpu as pltpu
from jax.experimental.pallas import tpu_sc as plsc

_E = 8
_H = 768
_I = 1536
_T = 2048
_BM = 128
_APAD = _T * 2 + _E * _BM
_NBLK = _APAD // _BM
_NW = 16
_TPW = _T // _NW
_LANES = 16


def _router_body(x_ref, rw_ref, logits_ref):
    logits_ref[...] = jnp.dot(x_ref[...], rw_ref[...],
                              preferred_element_type=jnp.float32)


def _splat(x):
    return jnp.full((_LANES,), x, jnp.int32)


def _dyn_gather(vec, idx):
    dnums = lax.GatherDimensionNumbers(
        offset_dims=(), collapsed_slice_dims=(0,), start_index_map=(0,))
    return lax.gather(vec, idx[:, None], dnums, slice_sizes=(1,),
                      mode=lax.GatherScatterMode.PROMISE_IN_BOUNDS)


def _route_body(logits_hbm, e1_hbm, e2_hbm, w0_hbm, w1_hbm, cnts_hbm, slab,
                e1b, e2b, w0b, w1b, cntv):
    nw = 32
    tpw = _T // nw
    wid = lax.axis_index("s") * 2 + lax.axis_index("c")
    lanes = lax.iota(jnp.int32, _LANES)
    zero_i = jnp.zeros((_LANES,), jnp.int32)

    pltpu.sync_copy(logits_hbm.at[pl.ds(wid * (tpw * _E), tpw * _E)], slab)
    cnt = zero_i
    for c in range(tpw // _LANES):
        tloc = lanes + c * _LANES
        m1 = jnp.full((_LANES,), -jnp.inf, jnp.float32)
        m2 = m1
        i1 = zero_i
        i2 = zero_i
        for e in range(_E):
            v = plsc.load_gather(slab, [tloc * _E + e])
            gt1 = v > m1
            gt2 = v > m2
            i2 = jnp.where(gt1, i1, jnp.where(gt2, e, i2))
            m2 = jnp.where(gt1, m1, jnp.where(gt2, v, m2))
            i1 = jnp.where(gt1, e, i1)
            m1 = jnp.where(gt1, v, m1)
        w0 = 1.0 / (1.0 + jnp.exp(m2 - m1))
        sl = pl.ds(c * _LANES, _LANES)
        e1b[sl] = i1
        e2b[sl] = i2
        w0b[sl] = w0
        w1b[sl] = 1.0 - w0
        for e in range(_E):
            pc = plsc.all_reduce_population_count(i1 == e)
            pc = pc + plsc.all_reduce_population_count(i2 == e)
            cnt = cnt + jnp.where(lanes == e, pc, 0)
    cntv[...] = cnt
    pltpu.sync_copy(e1b, e1_hbm.at[pl.ds(wid * tpw, tpw)])
    pltpu.sync_copy(e2b, e2_hbm.at[pl.ds(wid * tpw, tpw)])
    pltpu.sync_copy(w0b, w0_hbm.at[pl.ds(wid * tpw, tpw)])
    pltpu.sync_copy(w1b, w1_hbm.at[pl.ds(wid * tpw, tpw)])
    pltpu.sync_copy(cntv, cnts_hbm.at[pl.ds(wid * _LANES, _LANES)])


def _dispatch_body(cnts_hbm, e1_hbm, e2_hbm, w0_hbm, w1_hbm, x_hbm, xs_hbm,
                   w2d_hbm, pos0_hbm, pos1_hbm, be_hbm, ba_hbm, callv, e1b,
                   e2b, w0b, w1b, idxb, p0b, p1b, zidxb, xbuf, wbuf, zbuf,
                   bev, bav, sem, rsem):
    nw = 32
    tpw = _T // nw
    nck = tpw // _LANES
    wid = lax.axis_index("s") * 2 + lax.axis_index("c")
    lanes = lax.iota(jnp.int32, _LANES)
    zero_i = jnp.zeros((_LANES,), jnp.int32)

    pltpu.sync_copy(cnts_hbm, callv)
    pltpu.sync_copy(e1_hbm.at[pl.ds(wid * tpw, tpw)], e1b)
    pltpu.sync_copy(e2_hbm.at[pl.ds(wid * tpw, tpw)], e2b)
    pltpu.sync_copy(w0_hbm.at[pl.ds(wid * tpw, tpw)], w0b)
    pltpu.sync_copy(w1_hbm.at[pl.ds(wid * tpw, tpw)], w1b)

    acc = zero_i
    before = zero_i
    for w2 in range(nw):
        row = callv[pl.ds(w2 * _LANES, _LANES)]
        before = jnp.where(_splat(w2) == _splat(wid), acc, before)
        acc = acc + row
    tot = acc
    pad_tot = ((tot + (_BM - 1)) // _BM) * _BM
    off_pad = plsc.cumsum(pad_tot) - pad_tot
    base_vec = off_pad + before
    nb = pad_tot // _BM
    ends = plsc.cumsum(nb)
    used = jnp.sum(jnp.where(lanes < _E, nb, 0))

    @pl.when(wid == 0)
    def _blocks():
        last_e = jnp.sum(jnp.where((ends <= used - 1) & (lanes < _E), 1, 0))
        for c in range(48 // _LANES):
            bv = lanes + c * _LANES
            raw = zero_i
            for e in range(_E):
                ends_e = _dyn_gather(ends, _splat(e))
                raw = raw + jnp.where(ends_e <= bv, 1, 0)
            active = bv < used
            bev[pl.ds(c * _LANES, _LANES)] = jnp.where(active, raw, last_e)
            bav[pl.ds(c * _LANES, _LANES)] = active.astype(jnp.int32)
        pltpu.sync_copy(bev, be_hbm)
        pltpu.sync_copy(bav, ba_hbm)

    cnt_run = zero_i
    for cl in range(nck):
        lsl = pl.ds(cl * _LANES, _LANES)
        for s in range(2):
            ev = (e1b if s == 0 else e2b)[lsl]
            g = _dyn_gather(cnt_run, ev)
            prefix = zero_i
            ccnt = zero_i
            for e in range(_E):
                selm = ev == e
                pcs = plsc.cumsum(selm.astype(jnp.int32))
                prefix = jnp.where(selm, pcs - 1, prefix)
                ccnt = ccnt + jnp.where(
                    lanes == e, plsc.all_reduce_population_count(selm), 0)
            pos = _dyn_gather(base_vec, ev) + g + prefix
            cnt_run = cnt_run + ccnt
            idxb[cl * 2 + s, :] = pos
            if s == 0:
                p0b[lsl] = pos
            else:
                p1b[lsl] = pos
    pltpu.sync_copy(p0b, pos0_hbm.at[pl.ds(wid * tpw, tpw)])
    pltpu.sync_copy(p1b, pos1_hbm.at[pl.ds(wid * tpw, tpw)])

    for k in range(_LANES):
        for kk in range(128 // _LANES):
            zbuf[k, pl.ds(kk * _LANES, _LANES)] = jnp.zeros((_LANES,),
                                                            jnp.float32)
    for k in range(_BM // _LANES):
        j = lanes + k * _LANES
        dump = _APAD + lanes
        myexp = _splat(jnp.minimum(wid, _E - 1))
        cnt_e = _dyn_gather(tot, myexp)
        pad_e = _dyn_gather(pad_tot, myexp)
        off_e = _dyn_gather(off_pad, myexp)
        padidx = jnp.where(cnt_e == pad_e, dump,
                           off_e + jnp.minimum(cnt_e + j, pad_e - 1))
        zidxb[k, :] = jnp.where(_splat(wid) < _E, padidx, dump)
    wdescs = []
    for k in range(_BM // _LANES):
        wdescs.append(pltpu.async_copy(zbuf, w2d_hbm.at[zidxb.at[k]], sem))

    rdescs = []
    for cl in range(nck):
        rdescs.append(pltpu.async_copy(
            x_hbm.at[pl.ds(wid * tpw + cl * _LANES, _LANES)], xbuf.at[cl],
            rsem))
    for cl in range(nck):
        rdescs[cl].wait()
        for s in range(2):
            wv = (w0b if s == 0 else w1b)[pl.ds(cl * _LANES, _LANES)]
            for r in range(_LANES):
                wr = _dyn_gather(wv, _splat(r))
                wbuf[cl * 2 + s, r, pl.ds(0, _LANES)] = wr
            wdescs.append(pltpu.async_copy(
                xbuf.at[cl], xs_hbm.at[idxb.at[cl * 2 + s]], sem))
            wdescs.append(pltpu.async_copy(
                wbuf.at[cl * 2 + s], w2d_hbm.at[idxb.at[cl * 2 + s]], sem))
    for d in wdescs:
        d.wait()


def _gmm_body(be_ref, ba_ref, xs_ref, wg_ref, wu_ref, wd_ref, wcol_ref,
              out_ref):
    i = pl.program_id(0)

    @pl.when(ba_ref[i] == 1)
    def _active():
        xb = xs_ref[...].astype(jnp.bfloat16)
        wg16 = wg_ref[0].astype(jnp.bfloat16)
        wu16 = wu_ref[0].astype(jnp.bfloat16)
        wd16 = wd_ref[0].astype(jnp.bfloat16)
        g = jnp.dot(xb, wg16, preferred_element_type=jnp.float32)
        u = jnp.dot(xb, wu16, preferred_element_type=jnp.float32)
        act = (g * jax.nn.sigmoid(g) * u).astype(jnp.bfloat16)
        ob = jnp.dot(act, wd16, preferred_element_type=jnp.float32)
        out_ref[...] = ob * wcol_ref[:, 0:1]

    @pl.when(ba_ref[i] == 0)
    def _inactive():
        out_ref[...] = jnp.zeros_like(out_ref)


def _combine_body(os_hbm, p03_hbm, p13_hbm, out_hbm, idxA, idxB, bufA, bufB,
                  gsem, wsem):
    nw2 = 32
    tpw = _T // nw2
    ck = _LANES
    nck = tpw // ck
    wid = lax.axis_index("s") * 2 + lax.axis_index("c")
    pltpu.sync_copy(p03_hbm.at[wid], idxA)
    pltpu.sync_copy(p13_hbm.at[wid], idxB)
    gd = {}
    for c in range(2):
        gd[(c, 0)] = pltpu.async_copy(os_hbm.at[idxA.at[c]], bufA.at[c % 2],
                                      gsem)
        gd[(c, 1)] = pltpu.async_copy(os_hbm.at[idxB.at[c]], bufB.at[c % 2],
                                      gsem)
    wdescs = []
    for c in range(nck):
        gd[(c, 0)].wait()
        gd[(c, 1)].wait()

        def _add(r, carry, _c=c):
            for k in range(_H // _LANES):
                sl = pl.ds(k * _LANES, _LANES)
                bufA[_c % 2, r, sl] = (bufA[_c % 2, r, sl]
                                       + bufB[_c % 2, r, sl])
            return carry
        lax.fori_loop(0, ck, _add, 0)
        wdescs.append(pltpu.async_copy(
            bufA.at[c % 2], out_hbm.at[pl.ds(wid * tpw + c * ck, ck)], wsem))
        if c + 2 < nck:
            wdescs[c].wait()
            wdescs[c] = None
            gd[(c + 2, 0)] = pltpu.async_copy(os_hbm.at[idxA.at[c + 2]],
                                              bufA.at[c % 2], gsem)
            gd[(c + 2, 1)] = pltpu.async_copy(os_hbm.at[idxB.at[c + 2]],
                                              bufB.at[c % 2], gsem)
    for d in wdescs:
        if d is not None:
            d.wait()


_SC_PARAMS = pltpu.CompilerParams(needs_layout_passes=False)
_MESH1 = plsc.VectorSubcoreMesh(core_axis_name="c", subcore_axis_name="s",
                                num_cores=1)
_MESH2 = plsc.VectorSubcoreMesh(core_axis_name="c", subcore_axis_name="s",
                                num_cores=2)


def _make_router():
    return pl.pallas_call(
        _router_body,
        in_specs=[pl.BlockSpec((_T, _H), lambda: (0, 0)),
                  pl.BlockSpec((_H, _E), lambda: (0, 0))],
        out_specs=pl.BlockSpec((_T, _E), lambda: (0, 0)),
        out_shape=jax.ShapeDtypeStruct((_T, _E), jnp.float32),
    )


def _make_route():
    return pl.kernel(
        _route_body,
        compiler_params=_SC_PARAMS,
        out_type=[
            jax.ShapeDtypeStruct((_T,), jnp.int32),
            jax.ShapeDtypeStruct((_T,), jnp.int32),
            jax.ShapeDtypeStruct((_T,), jnp.float32),
            jax.ShapeDtypeStruct((_T,), jnp.float32),
            jax.ShapeDtypeStruct((32 * _LANES,), jnp.int32),
        ],
        mesh=_MESH2,
        scratch_types=[
            pltpu.VMEM((_T // 32 * _E,), jnp.float32),
            pltpu.VMEM((64,), jnp.int32),
            pltpu.VMEM((64,), jnp.int32),
            pltpu.VMEM((64,), jnp.float32),
            pltpu.VMEM((64,), jnp.float32),
            pltpu.VMEM((_LANES,), jnp.int32),
        ],
    )


def _make_dispatch():
    return pl.kernel(
        _dispatch_body,
        compiler_params=_SC_PARAMS,
        out_type=[
            jax.ShapeDtypeStruct((_APAD, _H), jnp.float32),
            jax.ShapeDtypeStruct((_APAD + 128, 128), jnp.float32),
            jax.ShapeDtypeStruct((_T,), jnp.int32),
            jax.ShapeDtypeStruct((_T,), jnp.int32),
            jax.ShapeDtypeStruct((48,), jnp.int32),
            jax.ShapeDtypeStruct((48,), jnp.int32),
        ],
        mesh=_MESH2,
        scratch_types=[
            pltpu.VMEM((32 * _LANES,), jnp.int32),
            pltpu.VMEM((64,), jnp.int32),
            pltpu.VMEM((64,), jnp.int32),
            pltpu.VMEM((64,), jnp.float32),
            pltpu.VMEM((64,), jnp.float32),
            pltpu.VMEM((8, _LANES), jnp.int32),
            pltpu.VMEM((64,), jnp.int32),
            pltpu.VMEM((64,), jnp.int32),
            pltpu.VMEM((_BM // _LANES, _LANES), jnp.int32),
            pltpu.VMEM((4, _LANES, _H), jnp.float32),
            pltpu.VMEM((8, _LANES, 128), jnp.float32),
            pltpu.VMEM((_LANES, 128), jnp.float32),
            pltpu.VMEM((48,), jnp.int32),
            pltpu.VMEM((48,), jnp.int32),
            pltpu.SemaphoreType.DMA,
            pltpu.SemaphoreType.DMA,
        ],
    )


def _make_gmm():
    return pl.pallas_call(
        _gmm_body,
        grid_spec=pltpu.PrefetchScalarGridSpec(
            num_scalar_prefetch=2,
            grid=(_NBLK,),
            in_specs=[
                pl.BlockSpec((_BM, _H), lambda i, be, ba: (i, 0)),
                pl.BlockSpec((1, _H, _I), lambda i, be, ba: (be[i], 0, 0)),
                pl.BlockSpec((1, _H, _I), lambda i, be, ba: (be[i], 0, 0)),
                pl.BlockSpec((1, _I, _H), lambda i, be, ba: (be[i], 0, 0)),
                pl.BlockSpec((_BM, 128), lambda i, be, ba: (i, 0)),
            ],
            out_specs=pl.BlockSpec((_BM, _H), lambda i, be, ba: (i, 0)),
        ),
        out_shape=jax.ShapeDtypeStruct((_APAD, _H), jnp.float32),
    )


def _make_combine():
    return pl.kernel(
        _combine_body,
        compiler_params=_SC_PARAMS,
        out_type=jax.ShapeDtypeStruct((_T, _H), jnp.float32),
        mesh=_MESH2,
        scratch_types=[
            pltpu.VMEM((4, _LANES), jnp.int32),
            pltpu.VMEM((4, _LANES), jnp.int32),
            pltpu.VMEM((2, _LANES, _H), jnp.float32),
            pltpu.VMEM((2, _LANES, _H), jnp.float32),
            pltpu.SemaphoreType.DMA,
            pltpu.SemaphoreType.DMA,
        ],
    )


def kernel(hidden_states, router_w, w_gate, w_up, w_down):
    shape = hidden_states.shape
    x = hidden_states.reshape(-1, _H)
    logits = _make_router()(x, router_w)
    e1, e2, w0, w1, cnts = _make_route()(logits.reshape(-1))
    x_sorted, w2d, pos0, pos1, blk_e, blk_a = _make_dispatch()(
        cnts, e1, e2, w0, w1, x)
    out_sorted = _make_gmm()(blk_e, blk_a, x_sorted, w_gate, w_up, w_down,
                             w2d[:_APAD])
    p03 = pos0.reshape(32, 4, _LANES)
    p13 = pos1.reshape(32, 4, _LANES)
    out = _make_combine()(out_sorted, p03, p13)
    return out.reshape(shape), logits
